# baseline (speedup 1.0000x reference)
import jax
import jax.numpy as jnp
from jax import lax
from jax.experimental import pallas as pl
from jax.experimental.pallas import tpu as pltpu
from jax.experimental.pallas import tpu_sc as plsc

NC = 2
NS = 16
L = 16
NW = NC * NS
BATCH = 16384
D = 64
W = 2 * D
NROWS = 1000000
HALF = NROWS // 2
BPW = BATCH // NW
KC = 128
NCHUNK = BPW // KC
GPC = KC // L


def _body(ids_hbm, bigA_hbm, bigB_hbm, out_hbm,
          ids_v, uid_v, iid_v, uidA_v, uidB_v, iidA_v, iidB_v,
          rowsUA_v, rowsUB_v, rowsIA_v, rowsIB_v, out_v, sem_u, sem_i):
    wid = lax.axis_index("s") * NC + lax.axis_index("c")
    base = pl.multiple_of(wid * BPW, BPW)

    pltpu.sync_copy(ids_hbm.at[pl.ds(base * 2, BPW * 2)], ids_v)

    lane = lax.iota(jnp.int32, L)
    lane2 = lane * 2
    hi = jnp.int32(HALF - 1)
    zero = jnp.int32(0)
    for c in range(NCHUNK):
        for g in range(GPC):
            sl = pl.ds(g * L, L)
            rows = lane2 + (c * KC + g * L) * 2
            u = plsc.load_gather(ids_v, [rows])
            i = plsc.load_gather(ids_v, [rows + 1])
            uid_v[c, sl] = u
            iid_v[c, sl] = i
            uidA_v[c, sl] = jnp.minimum(u, hi)
            uidB_v[c, sl] = jnp.maximum(u - HALF, zero)
            iidA_v[c, sl] = jnp.minimum(i, hi)
            iidB_v[c, sl] = jnp.maximum(i - HALF, zero)

    for c in range(NCHUNK):
        cps = (
            pltpu.async_copy(bigA_hbm.at[uidA_v.at[c]], rowsUA_v, sem_u),
            pltpu.async_copy(bigB_hbm.at[uidB_v.at[c]], rowsUB_v, sem_u),
            pltpu.async_copy(bigA_hbm.at[iidA_v.at[c]], rowsIA_v, sem_i),
            pltpu.async_copy(bigB_hbm.at[iidB_v.at[c]], rowsIB_v, sem_i),
        )
        for cp in cps:
            cp.wait()

        def group_body(g, carry):
            goff = pl.multiple_of(g * L, L)
            rows = lane + goff
            inA_u = uid_v[c, pl.ds(goff, L)] < HALF
            inA_i = iid_v[c, pl.ds(goff, L)] < HALF
            acc = jnp.zeros((L,), jnp.float32)
            for d in range(D):
                cu = jnp.full((L,), d, jnp.int32)
                ci = jnp.full((L,), D + d, jnp.int32)
                uval = jnp.where(inA_u,
                                 plsc.load_gather(rowsUA_v, [rows, cu]),
                                 plsc.load_gather(rowsUB_v, [rows, cu]))
                ival = jnp.where(inA_i,
                                 plsc.load_gather(rowsIA_v, [rows, ci]),
                                 plsc.load_gather(rowsIB_v, [rows, ci]))
                acc = acc + uval * ival
            out_v[pl.ds(c * KC + goff, L)] = acc
            return carry

        lax.fori_loop(0, GPC, group_body, 0)

    pltpu.sync_copy(out_v, out_hbm.at[pl.ds(base, BPW)])


def kernel(inputs, user_table, item_table):
    bigA = jnp.concatenate([user_table[:HALF], item_table[:HALF]], axis=1)
    bigB = jnp.concatenate([user_table[HALF:], item_table[HALF:]], axis=1)
    mesh = plsc.VectorSubcoreMesh(core_axis_name="c", subcore_axis_name="s",
                                  num_cores=NC, num_subcores=NS)
    f = pl.kernel(
        _body,
        out_type=jax.ShapeDtypeStruct((BATCH,), jnp.float32),
        mesh=mesh,
        compiler_params=pltpu.CompilerParams(needs_layout_passes=False),
        scratch_types=[
            pltpu.VMEM((BPW * 2,), jnp.int32),
            pltpu.VMEM((NCHUNK, KC), jnp.int32),
            pltpu.VMEM((NCHUNK, KC), jnp.int32),
            pltpu.VMEM((NCHUNK, KC), jnp.int32),
            pltpu.VMEM((NCHUNK, KC), jnp.int32),
            pltpu.VMEM((NCHUNK, KC), jnp.int32),
            pltpu.VMEM((NCHUNK, KC), jnp.int32),
            pltpu.VMEM((KC, W), jnp.float32),
            pltpu.VMEM((KC, W), jnp.float32),
            pltpu.VMEM((KC, W), jnp.float32),
            pltpu.VMEM((KC, W), jnp.float32),
            pltpu.VMEM((BPW,), jnp.float32),
            pltpu.SemaphoreType.DMA,
            pltpu.SemaphoreType.DMA,
        ],
    )
    return f(inputs.reshape(-1), bigA, bigB)


# speedup vs baseline: 1.8907x; 1.8907x over previous
import jax
import jax.numpy as jnp
from jax import lax
from jax.experimental import pallas as pl
from jax.experimental.pallas import tpu as pltpu
from jax.experimental.pallas import tpu_sc as plsc

NC = 2
NS = 16
L = 16
NW = NC * NS
BATCH = 16384
D = 64
W = 2 * D
BPW = BATCH // NW
KC = 128
NCHUNK = BPW // KC
GPC = KC // L


def _body(ids_hbm, user2_hbm, item2_hbm, out_hbm,
          ids_v, uid_v, iid_v, upar_v, ipar_v,
          urows_v, irows_v, out_v, sem_u, sem_i):
    wid = lax.axis_index("s") * NC + lax.axis_index("c")
    base = pl.multiple_of(wid * BPW, BPW)

    pltpu.sync_copy(ids_hbm.at[pl.ds(base * 2, BPW * 2)], ids_v)

    lane = lax.iota(jnp.int32, L)
    lane2 = lane * 2
    for c in range(NCHUNK):
        for g in range(GPC):
            sl = pl.ds(g * L, L)
            rows = lane2 + (c * KC + g * L) * 2
            u = plsc.load_gather(ids_v, [rows])
            i = plsc.load_gather(ids_v, [rows + 1])
            uid_v[c, sl] = u >> 1
            iid_v[c, sl] = i >> 1
            upar_v[c, sl] = (u & 1) << 6
            ipar_v[c, sl] = (i & 1) << 6

    def fire(c):
        cp_u = pltpu.async_copy(user2_hbm.at[uid_v.at[c]],
                                urows_v.at[c % 2], sem_u)
        cp_i = pltpu.async_copy(item2_hbm.at[iid_v.at[c]],
                                irows_v.at[c % 2], sem_i)
        return cp_u, cp_i

    pending = fire(0)
    for c in range(NCHUNK):
        pending[0].wait()
        pending[1].wait()
        if c + 1 < NCHUNK:
            pending = fire(c + 1)
        ub = urows_v.at[c % 2]
        ib = irows_v.at[c % 2]

        def group_body(g, carry):
            goff = pl.multiple_of(g * L, L)
            rows = lane + goff
            ucol = upar_v[c, pl.ds(goff, L)]
            icol = ipar_v[c, pl.ds(goff, L)]
            acc = jnp.zeros((L,), jnp.float32)
            for d in range(D):
                acc = acc + (plsc.load_gather(ub, [rows, ucol + d]) *
                             plsc.load_gather(ib, [rows, icol + d]))
            out_v[pl.ds(c * KC + goff, L)] = acc
            return carry

        lax.fori_loop(0, GPC, group_body, 0)

    pltpu.sync_copy(out_v, out_hbm.at[pl.ds(base, BPW)])


def kernel(inputs, user_table, item_table):
    user2 = user_table.reshape(-1, W)
    item2 = item_table.reshape(-1, W)
    mesh = plsc.VectorSubcoreMesh(core_axis_name="c", subcore_axis_name="s",
                                  num_cores=NC, num_subcores=NS)
    f = pl.kernel(
        _body,
        out_type=jax.ShapeDtypeStruct((BATCH,), jnp.float32),
        mesh=mesh,
        compiler_params=pltpu.CompilerParams(needs_layout_passes=False),
        scratch_types=[
            pltpu.VMEM((BPW * 2,), jnp.int32),
            pltpu.VMEM((NCHUNK, KC), jnp.int32),
            pltpu.VMEM((NCHUNK, KC), jnp.int32),
            pltpu.VMEM((NCHUNK, KC), jnp.int32),
            pltpu.VMEM((NCHUNK, KC), jnp.int32),
            pltpu.VMEM((2, KC, W), jnp.float32),
            pltpu.VMEM((2, KC, W), jnp.float32),
            pltpu.VMEM((BPW,), jnp.float32),
            pltpu.SemaphoreType.DMA,
            pltpu.SemaphoreType.DMA,
        ],
    )
    return f(inputs.reshape(-1), user2, item2)


# speedup vs baseline: 2.3080x; 1.2207x over previous
import jax
import jax.numpy as jnp
from jax import lax
from jax.experimental import pallas as pl
from jax.experimental.pallas import tpu as pltpu
from jax.experimental.pallas import tpu_sc as plsc

NC = 2
NS = 16
L = 16
NW = NC * NS
BATCH = 16384
D = 64
W = 2 * D
BPW = BATCH // NW
KC = 128
NCHUNK = BPW // KC
GPC = KC // L


def _body(ids_hbm, big_hbm, out_hbm,
          ids_v, uid_v, iid_v, urows_v, irows_v, out_v, sem_u, sem_i):
    wid = lax.axis_index("s") * NC + lax.axis_index("c")
    base = pl.multiple_of(wid * BPW, BPW)

    pltpu.sync_copy(ids_hbm.at[pl.ds(base * 2, BPW * 2)], ids_v)

    lane = lax.iota(jnp.int32, L)
    lane2 = lane * 2
    for c in range(NCHUNK):
        for g in range(GPC):
            rows = lane2 + (c * KC + g * L) * 2
            uid_v[c, pl.ds(g * L, L)] = plsc.load_gather(ids_v, [rows])
            iid_v[c, pl.ds(g * L, L)] = plsc.load_gather(ids_v, [rows + 1])

    def fire(c):
        cp_u = pltpu.async_copy(big_hbm.at[uid_v.at[c]], urows_v.at[c % 2],
                                sem_u)
        cp_i = pltpu.async_copy(big_hbm.at[iid_v.at[c]], irows_v.at[c % 2],
                                sem_i)
        return cp_u, cp_i

    pending = fire(0)
    for c in range(NCHUNK):
        pending[0].wait()
        pending[1].wait()
        if c + 1 < NCHUNK:
            pending = fire(c + 1)
        ub = urows_v.at[c % 2]
        ib = irows_v.at[c % 2]

        def group_body(g, carry):
            goff = pl.multiple_of(g * L, L)
            rows = lane + goff
            acc = jnp.zeros((L,), jnp.float32)
            for d in range(D):
                cu = jnp.full((L,), d, jnp.int32)
                ci = jnp.full((L,), D + d, jnp.int32)
                acc = acc + (plsc.load_gather(ub, [rows, cu]) *
                             plsc.load_gather(ib, [rows, ci]))
            out_v[pl.ds(c * KC + goff, L)] = acc
            return carry

        lax.fori_loop(0, GPC, group_body, 0)

    pltpu.sync_copy(out_v, out_hbm.at[pl.ds(base, BPW)])


def kernel(inputs, user_table, item_table):
    one = (1 - inputs[0, 0] * 0).astype(jnp.float32)
    big = jnp.concatenate([user_table, item_table], axis=1) * one
    mesh = plsc.VectorSubcoreMesh(core_axis_name="c", subcore_axis_name="s",
                                  num_cores=NC, num_subcores=NS)
    f = pl.kernel(
        _body,
        out_type=jax.ShapeDtypeStruct((BATCH,), jnp.float32),
        mesh=mesh,
        compiler_params=pltpu.CompilerParams(needs_layout_passes=False),
        scratch_types=[
            pltpu.VMEM((BPW * 2,), jnp.int32),
            pltpu.VMEM((NCHUNK, KC), jnp.int32),
            pltpu.VMEM((NCHUNK, KC), jnp.int32),
            pltpu.VMEM((2, KC, W), jnp.float32),
            pltpu.VMEM((2, KC, W), jnp.float32),
            pltpu.VMEM((BPW,), jnp.float32),
            pltpu.SemaphoreType.DMA,
            pltpu.SemaphoreType.DMA,
        ],
    )
    return f(inputs.reshape(-1), big)
